# Initial kernel scaffold; baseline (speedup 1.0000x reference)
import jax
import jax.numpy as jnp
from jax.experimental import pallas as pl


def kernel(p, x, o, params):
    raise NotImplementedError("write your pallas kernel here")


# speedup vs baseline: 3.5316x; 3.5316x over previous
import functools

import jax
import jax.numpy as jnp
from jax.experimental import pallas as pl

NSAMPLE = 8
SHARE = 8
EPS = 1e-5


def _bn(x, p):
    return (x - p["m"]) / jnp.sqrt(p["v"] + EPS) * p["g"] + p["b"]


def _knn_body(q_ref, pt_ref, pn_ref, out_ref, *, rows, npad):
    q = q_ref[...]
    qn = jnp.sum(q * q, axis=1, keepdims=True)
    pt = pt_ref[...]
    pn = pn_ref[...]
    dot = jax.lax.dot_general(
        q.astype(jnp.bfloat16), pt, (((1,), (0,)), ((), ())),
        preferred_element_type=jnp.float32)
    d = qn + pn - 2.0 * dot
    iota = jax.lax.broadcasted_iota(jnp.int32, (rows, npad), 1)
    big_f = jnp.float32(3.0e38)
    big_i = jnp.int32(2**31 - 1)
    for k in range(NSAMPLE):
        m = jnp.min(d, axis=1, keepdims=True)
        am = jnp.min(jnp.where(d == m, iota, big_i), axis=1, keepdims=True)
        out_ref[k, :] = am[:, 0]
        d = jnp.where(iota == am, big_f, d)


@functools.partial(jax.jit, static_argnames=("rows",))
def _knn_topk(p, rows=128):
    n = p.shape[0]
    npad = -(-n // 128) * 128
    nqpad = -(-n // rows) * rows
    p8 = jnp.zeros((npad, 8), jnp.float32).at[:n, :3].set(p)
    pt = p8.T.astype(jnp.bfloat16)
    pn = jnp.sum(p * p, axis=1)
    pn = jnp.full((1, npad), 1e30, jnp.float32).at[0, :n].set(pn)
    qpad = jnp.zeros((nqpad, 8), jnp.float32).at[:n, :3].set(p)

    idx_t = pl.pallas_call(
        functools.partial(_knn_body, rows=rows, npad=npad),
        grid=(nqpad // rows,),
        in_specs=[
            pl.BlockSpec((rows, 8), lambda i: (i, 0)),
            pl.BlockSpec((8, npad), lambda i: (0, 0)),
            pl.BlockSpec((1, npad), lambda i: (0, 0)),
        ],
        out_specs=pl.BlockSpec((NSAMPLE, rows), lambda i: (0, i)),
        out_shape=jax.ShapeDtypeStruct((NSAMPLE, nqpad), jnp.int32),
    )(qpad, pt, pn)
    return idx_t[:, :n].T


def kernel(p, x, o, params):
    del o
    prm = params
    h = jax.nn.relu(_bn(x @ prm["td_W"], prm["td_bn"]))
    identity = h
    h1 = jax.nn.relu(_bn(h @ prm["lin1_W"], prm["bn1"]))
    xq = h1 @ prm["Wq"] + prm["bq"]
    xk = h1 @ prm["Wk"] + prm["bk"]
    xv = h1 @ prm["Wv"] + prm["bv"]
    idx = _knn_topk(p)
    rel = p[idx] - p[:, None, :]
    gk = xk[idx]
    gv = xv[idx]
    p_r = rel @ prm["P1"] + prm["P1b"]
    p_r = jax.nn.relu(_bn(p_r, prm["bnp"]))
    p_r = p_r @ prm["P2"] + prm["P2b"]
    w = gk - xq[:, None, :] + p_r
    w = jax.nn.relu(_bn(w, prm["bnw1"]))
    w = w @ prm["W1"] + prm["W1b"]
    w = jax.nn.relu(_bn(w, prm["bnw2"]))
    w = w @ prm["W2"] + prm["W2b"]
    w = jax.nn.softmax(w, axis=1)
    n, ns, c = gv.shape
    agg = ((gv + p_r).reshape(n, ns, SHARE, c // SHARE)
           * w[:, :, None, :]).sum(1).reshape(n, c)
    h2 = jax.nn.relu(_bn(agg, prm["bn2"]))
    h3 = _bn(h2 @ prm["lin3_W"], prm["bn3"])
    return jax.nn.relu(h3 + identity)
